# baseline (speedup 1.0000x reference)
import jax
import jax.numpy as jnp
from jax import lax
from jax.experimental import pallas as pl
from jax.experimental.pallas import tpu as pltpu

B, N, D = 128, 576, 768
CB = 4
NSTEP = B // CB
NBUF = 8
PF = NBUF // 2


def _ring_kernel(x_hbm, t_hbm, o_hbm, tbl, *rest):
    slots = rest[:NBUF]
    ts = rest[NBUF]
    sins = rest[NBUF + 1:2 * NBUF + 1]
    souts = rest[2 * NBUF + 1:]

    pltpu.async_copy(t_hbm, tbl, ts).wait()

    def src(s):
        return x_hbm.at[pl.ds(s * CB, CB)]

    def dst(s):
        return o_hbm.at[pl.ds(s * CB, CB)]

    def add(slot):
        slot[...] = slot[...] + tbl[...][None, :, :]

    def phase(s, slot_of_s, dyn=False):
        i = slot_of_s % NBUF
        pltpu.make_async_copy(src(s), slots[i], sins[i]).wait()
        if dyn:
            j = (slot_of_s + PF) % NBUF
            pltpu.make_async_copy(slots[j], dst(s - PF), souts[j]).wait()
            pltpu.async_copy(src(s + PF), slots[j], sins[j])
        add(slots[i])
        pltpu.async_copy(slots[i], dst(s), souts[i])

    for s in range(PF):
        pltpu.async_copy(src(s), slots[s], sins[s])
    for s in range(PF):
        pltpu.make_async_copy(src(s), slots[s], sins[s]).wait()
        pltpu.async_copy(src(s + PF), slots[s + PF], sins[s + PF])
        add(slots[s])
        pltpu.async_copy(slots[s], dst(s), souts[s])

    def group(g, c):
        for k in range(NBUF):
            s = NBUF * g + PF + k
            phase(s, PF + k, dyn=True)
        return c

    lax.fori_loop(0, (NSTEP - 2 * PF) // NBUF, group, 0)

    for s in range(NSTEP - PF, NSTEP):
        i = s % NBUF
        pltpu.make_async_copy(src(s), slots[i], sins[i]).wait()
        pltpu.make_async_copy(slots[(i + PF) % NBUF], dst(s - PF),
                              souts[(i + PF) % NBUF]).wait()
        add(slots[i])
        pltpu.async_copy(slots[i], dst(s), souts[i])
    for s in range(NSTEP - PF, NSTEP):
        i = s % NBUF
        pltpu.make_async_copy(slots[i], dst(s), souts[i]).wait()


def kernel(encoded_patches, position_embedding):
    return pl.pallas_call(
        _ring_kernel,
        in_specs=[
            pl.BlockSpec(memory_space=pltpu.HBM),
            pl.BlockSpec(memory_space=pltpu.HBM),
        ],
        out_specs=pl.BlockSpec(memory_space=pltpu.HBM),
        out_shape=jax.ShapeDtypeStruct((B, N, D), jnp.float32),
        scratch_shapes=(
            [pltpu.VMEM((N, D), jnp.float32)]
            + [pltpu.VMEM((CB, N, D), jnp.float32) for _ in range(NBUF)]
            + [pltpu.SemaphoreType.DMA]
            + [pltpu.SemaphoreType.DMA for _ in range(2 * NBUF)]
        ),
        compiler_params=pltpu.CompilerParams(
            vmem_limit_bytes=62 * 1024 * 1024,
        ),
    )(encoded_patches, position_embedding)


# speedup vs baseline: 1.0084x; 1.0084x over previous
import jax
import jax.numpy as jnp
from jax.experimental import pallas as pl


def _add_kernel(x_ref, t_ref, o_ref):
    o_ref[...] = x_ref[...] + t_ref[...][None, :, :]


def kernel(encoded_patches, position_embedding):
    B, N, D = encoded_patches.shape
    BB = 8
    return pl.pallas_call(
        _add_kernel,
        grid=(B // BB,),
        in_specs=[
            pl.BlockSpec((BB, N, D), lambda i: (i, 0, 0)),
            pl.BlockSpec((N, D), lambda i: (0, 0)),
        ],
        out_specs=pl.BlockSpec((BB, N, D), lambda i: (i, 0, 0)),
        out_shape=jax.ShapeDtypeStruct((B, N, D), jnp.float32),
    )(encoded_patches, position_embedding)
